# Initial kernel scaffold; baseline (speedup 1.0000x reference)
import jax
import jax.numpy as jnp
from jax.experimental import pallas as pl


def kernel(new_ft, raw_ft, edge_index):
    raise NotImplementedError("write your pallas kernel here")


# speedup vs baseline: 2.6945x; 2.6945x over previous
import functools

import jax
import jax.numpy as jnp
from jax import lax
from jax.experimental import pallas as pl
from jax.experimental.pallas import tpu as pltpu
from jax.experimental.pallas import tpu_sc as plsc

N_CORES = 2
N_SUBCORES = 16
N_WORKERS = N_CORES * N_SUBCORES
LANES = 16

CHUNK = 64

_PERM_DNUMS = lax.GatherDimensionNumbers(
    offset_dims=(), collapsed_slice_dims=(0,), start_index_map=(0,))


def _lane_perm(x, idx):
  return lax.gather(x, idx[:, None], _PERM_DNUMS, (1,),
                    mode=lax.GatherScatterMode.PROMISE_IN_BOUNDS)


def _make_sc_kernel(n_nodes, n_edges, d_feat):
  assert n_edges % N_WORKERS == 0
  per_w = n_edges // N_WORKERS
  assert per_w % 8 == 0
  n_full = per_w // CHUNK
  tail = per_w - n_full * CHUNK
  d_vecs = d_feat // LANES
  groups = CHUNK // LANES

  mesh = plsc.VectorSubcoreMesh(core_axis_name="c", subcore_axis_name="s")

  @functools.partial(
      pl.kernel,
      out_type=jax.ShapeDtypeStruct((n_edges,), jnp.float32),
      mesh=mesh,
      scratch_types=[
          pltpu.VMEM((per_w,), jnp.int32),
          pltpu.VMEM((per_w,), jnp.int32),
          pltpu.VMEM((2, CHUNK, d_feat), jnp.float32),
          pltpu.VMEM((2, CHUNK, d_feat), jnp.float32),
          pltpu.VMEM((per_w,), jnp.float32),
          pltpu.SemaphoreType.DMA((2,)),
          pltpu.SemaphoreType.DMA((2,)),
      ],
  )
  def k(new_hbm, raw_hbm, src_hbm, dst_hbm, out_hbm,
        idx_u, idx_v, rows_u, rows_v, scores, sem_u, sem_v):
    wid = lax.axis_index("s") * N_CORES + lax.axis_index("c")
    lane = lax.iota(jnp.int32, LANES)
    base = pl.multiple_of(wid * per_w, 8)

    pltpu.sync_copy(src_hbm.at[pl.ds(base, per_w)], idx_u)
    pltpu.sync_copy(dst_hbm.at[pl.ds(base, per_w)], idx_v)

    def issue(c, b, size):
      off = pl.multiple_of(c * CHUNK, 8)
      cp_u = pltpu.async_copy(new_hbm.at[idx_u.at[pl.ds(off, size)]],
                              rows_u.at[b, pl.ds(0, size)], sem_u.at[b])
      cp_v = pltpu.async_copy(raw_hbm.at[idx_v.at[pl.ds(off, size)]],
                              rows_v.at[b, pl.ds(0, size)], sem_v.at[b])
      return cp_u, cp_v

    def wait(c, b, size):
      off = pl.multiple_of(c * CHUNK, 8)
      pltpu.make_async_copy(new_hbm.at[idx_u.at[pl.ds(off, size)]],
                            rows_u.at[b, pl.ds(0, size)], sem_u.at[b]).wait()
      pltpu.make_async_copy(raw_hbm.at[idx_v.at[pl.ds(off, size)]],
                            rows_v.at[b, pl.ds(0, size)], sem_v.at[b]).wait()

    def dot_group(b, ebase):
      stack = []
      for i in range(LANES):
        e = ebase + i
        vec = rows_u[b, e, pl.ds(0, LANES)] * rows_v[b, e, pl.ds(0, LANES)]
        for j in range(1, d_vecs):
          vec = vec + (rows_u[b, e, pl.ds(j * LANES, LANES)]
                       * rows_v[b, e, pl.ds(j * LANES, LANES)])
        lvl = 0
        while stack and stack[-1][0] == lvl:
          prev = stack.pop()[1]
          step = 1 << lvl
          perm = lane ^ step
          mask = (lane & step) == 0
          vec = jnp.where(mask, prev + _lane_perm(prev, perm),
                          vec + _lane_perm(vec, perm))
          lvl += 1
        stack.append((lvl, vec))
      return stack[0][1]

    def compute(c, b):
      sbase = pl.multiple_of(c * CHUNK, 8)

      def group_body(g, carry):
        gbase = pl.multiple_of(g * LANES, LANES)
        scores[pl.ds(sbase + gbase, LANES)] = dot_group(b, gbase)
        return carry

      lax.fori_loop(0, groups, group_body, 0, unroll=False)

    issue(0, 0, CHUNK)

    def chunk_body(t, carry):
      b = t % 2

      @pl.when(t + 1 < n_full)
      def _():
        issue(t + 1, 1 - b, CHUNK)

      wait(t, b, CHUNK)
      compute(t, b)
      return carry

    lax.fori_loop(0, n_full, chunk_body, 0, unroll=False)

    if tail:
      toff = pl.multiple_of(per_w - LANES, 8)
      cp_u = pltpu.async_copy(new_hbm.at[idx_u.at[pl.ds(toff, LANES)]],
                              rows_u.at[0, pl.ds(0, LANES)], sem_u.at[0])
      cp_v = pltpu.async_copy(raw_hbm.at[idx_v.at[pl.ds(toff, LANES)]],
                              rows_v.at[0, pl.ds(0, LANES)], sem_v.at[0])
      cp_u.wait()
      cp_v.wait()
      scores[pl.ds(toff, LANES)] = dot_group(0, 0)

    pltpu.sync_copy(scores, out_hbm.at[pl.ds(base, per_w)])

  return k


def kernel(new_ft, raw_ft, edge_index):
  n_nodes, d_feat = new_ft.shape
  n_edges = edge_index.shape[1]
  src = edge_index[0].astype(jnp.int32)
  dst = edge_index[1].astype(jnp.int32)
  k = _make_sc_kernel(n_nodes, n_edges, d_feat)
  score = k(new_ft, raw_ft, src, dst)
  return score.reshape(n_edges, 1)
